# baseline (speedup 1.0000x reference)
import jax
import jax.numpy as jnp
from jax import lax
from jax.experimental import pallas as pl
from jax.experimental.pallas import tpu as pltpu
from jax.experimental.pallas import tpu_sc as plsc

N_NODES = 100000
N_EDGES = 1600000
D_FEAT = 128
D_EDGE = 16
N_SEG = 256
N_CORES = 2
N_SUB = 16
N_WORKERS = N_CORES * N_SUB

NCH = 160
NCHUNKS = N_NODES // NCH
MAXN = 20
TC_TOT = N_EDGES // 128
TCC = 10
ECHUNKS = TC_TOT // TCC
MAXE = 40
EV = TCC * 128 // 16


def _hsum16(v):
    idx = lax.iota(jnp.int32, 16)
    for k in (8, 4, 2, 1):
        v = v + v.at[idx ^ k].get(mode="promise_in_bounds", unique_indices=True)
    return v[0]


def _sc_segsum(x, xb, ea4, eb):
    mesh = plsc.VectorSubcoreMesh(core_axis_name="c", subcore_axis_name="s")

    def body(x_hbm, xb_hbm, ea_hbm, eb_hbm, nout, eout,
             ebufA, ebufB, eidxA, eidxB, esemA, esemB,
             nbufA, nbufB, nidxA, nidxB, nsemA, nsemB,
             eacc, nacc):
        c = lax.axis_index("c")
        s = lax.axis_index("s")
        w = c * N_SUB + s
        zero16 = jnp.zeros((16,), jnp.float32)

        def zerow_e(i, _):
            for j in range(N_SEG // 16):
                eacc[i, pl.ds(j * 16, 16)] = zero16
            return 0
        lax.fori_loop(0, D_EDGE, zerow_e, 0)

        def zerow_n(i, _):
            for j in range(D_FEAT // 16):
                nacc[i, pl.ds(j * 16, 16)] = zero16
            return 0
        lax.fori_loop(0, N_SEG, zerow_n, 0)

        egbase = jnp.where(w < 2, w * 40, 80 + (w - 2) * 39)
        ecount = jnp.where(w < 2, 40, 39)

        def e_start(ci, buf, idx, sem):
            @pl.when(ci < ecount)
            def _():
                tc0 = (egbase + ci) * TCC
                pltpu.async_copy(ea_hbm.at[0, pl.ds(tc0, TCC)], buf.at[0], sem)
                pltpu.async_copy(ea_hbm.at[1, pl.ds(tc0, TCC)], buf.at[1], sem)
                pltpu.async_copy(eb_hbm.at[pl.ds(tc0 * 128, TCC * 128)], idx, sem)

        def e_wait(buf, idx, sem):
            pltpu.make_async_copy(ea_hbm.at[0, pl.ds(0, TCC)], buf.at[0], sem).wait()
            pltpu.make_async_copy(ea_hbm.at[1, pl.ds(0, TCC)], buf.at[1], sem).wait()
            pltpu.make_async_copy(eb_hbm.at[pl.ds(0, TCC * 128)], idx, sem).wait()

        def e_compute(buf, idx):
            first = idx[pl.ds(0, 16)][0]
            last = idx[pl.ds(TCC * 128 - 16, 16)][15]

            def per_seg(sg, _):
                def acc_v(v, vaccs):
                    iv = idx[pl.ds(v * 16, 16)]
                    m = iv == sg
                    tc = v // 8
                    lane0 = (v % 8) * 16
                    out = []
                    for tr in range(2):
                        for fr in range(8):
                            d = buf[tr, tc, fr, pl.ds(lane0, 16)]
                            out.append(vaccs[tr * 8 + fr]
                                       + jnp.where(m, d, 0.0))
                    return tuple(out)
                vaccs = lax.fori_loop(0, EV, acc_v,
                                      tuple(zero16 for _ in range(D_EDGE)))
                sga = (sg // 16) * 16
                lane_m = lax.iota(jnp.int32, 16) == (sg - sga)
                for f in range(D_EDGE):
                    tot = _hsum16(vaccs[f])
                    row = eacc[f, pl.ds(sga, 16)]
                    eacc[f, pl.ds(sga, 16)] = row + jnp.where(lane_m, tot, 0.0)
                return 0

            lax.fori_loop(first, last + 1, per_seg, 0)

        e_start(0, ebufA, eidxA, esemA)

        def e_body(ci2, _):
            ci0 = ci2 * 2
            e_start(ci0 + 1, ebufB, eidxB, esemB)
            @pl.when(ci0 < ecount)
            def _():
                e_wait(ebufA, eidxA, esemA)
                e_compute(ebufA, eidxA)
            e_start(ci0 + 2, ebufA, eidxA, esemA)
            @pl.when(ci0 + 1 < ecount)
            def _():
                e_wait(ebufB, eidxB, esemB)
                e_compute(ebufB, eidxB)
            return 0

        lax.fori_loop(0, MAXE // 2, e_body, 0)

        ngbase = jnp.where(w < 17, w * 20, 340 + (w - 17) * 19)
        ncount = jnp.where(w < 17, 20, 19)

        def n_start(ci, buf, idx, sem):
            @pl.when(ci < ncount)
            def _():
                rb = (ngbase + ci) * NCH
                pltpu.async_copy(x_hbm.at[pl.ds(rb, NCH)], buf, sem)
                pltpu.async_copy(xb_hbm.at[pl.ds(rb, NCH)], idx, sem)

        def n_wait(buf, idx, sem):
            pltpu.make_async_copy(x_hbm.at[pl.ds(0, NCH)], buf, sem).wait()
            pltpu.make_async_copy(xb_hbm.at[pl.ds(0, NCH)], idx, sem).wait()

        def n_compute(buf, idx):
            first = idx[pl.ds(0, 16)][0]
            last = idx[pl.ds(NCH - 16, 16)][15]

            def per_seg(sg, _):
                def acc_g(g, vaccs):
                    iv = idx[pl.ds(g * 16, 16)]
                    out = list(vaccs)
                    for rr in range(16):
                        fac = (iv[rr] == sg).astype(jnp.float32)
                        r = g * 16 + rr
                        for j in range(D_FEAT // 16):
                            d = buf[r, pl.ds(j * 16, 16)]
                            out[j] = out[j] + d * fac
                    return tuple(out)
                vaccs = lax.fori_loop(0, NCH // 16, acc_g,
                                      tuple(zero16 for _ in range(D_FEAT // 16)))
                for j in range(D_FEAT // 16):
                    col = nacc[sg, pl.ds(j * 16, 16)]
                    nacc[sg, pl.ds(j * 16, 16)] = col + vaccs[j]
                return 0

            lax.fori_loop(first, last + 1, per_seg, 0)

        n_start(0, nbufA, nidxA, nsemA)

        def n_body(ci2, _):
            ci0 = ci2 * 2
            n_start(ci0 + 1, nbufB, nidxB, nsemB)
            @pl.when(ci0 < ncount)
            def _():
                n_wait(nbufA, nidxA, nsemA)
                n_compute(nbufA, nidxA)
            n_start(ci0 + 2, nbufA, nidxA, nsemA)
            @pl.when(ci0 + 1 < ncount)
            def _():
                n_wait(nbufB, nidxB, nsemB)
                n_compute(nbufB, nidxB)
            return 0

        lax.fori_loop(0, MAXN // 2, n_body, 0)

        pltpu.sync_copy(nacc, nout.at[w])
        pltpu.sync_copy(eacc, eout.at[w])

    f = pl.kernel(
        body,
        out_type=(jax.ShapeDtypeStruct((N_WORKERS, N_SEG, D_FEAT), jnp.float32),
                  jax.ShapeDtypeStruct((N_WORKERS, D_EDGE, N_SEG), jnp.float32)),
        mesh=mesh,
        compiler_params=pltpu.CompilerParams(use_tc_tiling_on_sc=False),
        scratch_types=[
            pltpu.VMEM((2, TCC, 8, 128), jnp.float32),
            pltpu.VMEM((2, TCC, 8, 128), jnp.float32),
            pltpu.VMEM((TCC * 128,), jnp.int32),
            pltpu.VMEM((TCC * 128,), jnp.int32),
            pltpu.SemaphoreType.DMA,
            pltpu.SemaphoreType.DMA,
            pltpu.VMEM((NCH, D_FEAT), jnp.float32),
            pltpu.VMEM((NCH, D_FEAT), jnp.float32),
            pltpu.VMEM((NCH,), jnp.int32),
            pltpu.VMEM((NCH,), jnp.int32),
            pltpu.SemaphoreType.DMA,
            pltpu.SemaphoreType.DMA,
            pltpu.VMEM((D_EDGE, N_SEG), jnp.float32),
            pltpu.VMEM((N_SEG, D_FEAT), jnp.float32),
        ],
    )
    return f(x, xb, ea4, eb)


def _mlp(nparts, eparts, u, w0e, w0n, w0u, b0, w1, b1, wo, bo):
    def body(np_ref, ep_ref, u_ref, w0e_ref, w0n_ref, w0u_ref, b0_ref,
             w1_ref, b1_ref, wo_ref, bo_ref, o_ref):
        n = jnp.sum(np_ref[...], axis=0)
        e = jnp.sum(ep_ref[...], axis=0)
        he = lax.dot_general(e, w0e_ref[...], (((0,), (0,)), ((), ())),
                             preferred_element_type=jnp.float32)
        h = (he
             + jnp.dot(n, w0n_ref[...], preferred_element_type=jnp.float32)
             + jnp.dot(u_ref[...], w0u_ref[...], preferred_element_type=jnp.float32)
             + b0_ref[...])
        h = jnp.where(h >= 0, h, 0.2 * h)
        h = jnp.dot(h, w1_ref[...], preferred_element_type=jnp.float32) + b1_ref[...]
        h = jnp.where(h >= 0, h, 0.2 * h)
        o_ref[...] = (jnp.dot(h, wo_ref[...], preferred_element_type=jnp.float32)
                      + bo_ref[...])

    return pl.pallas_call(
        body,
        out_shape=jax.ShapeDtypeStruct((N_SEG, 128), jnp.float32),
    )(nparts, eparts, u, w0e, w0n, w0u, b0, w1, b1, wo, bo)


def kernel(x, edge_index, edge_attr, u, x_batch, edge_attr_batch,
           W0, b0, W1, b1, W_out, b_out):
    ea4 = edge_attr.T.reshape(2, 8, TC_TOT, 128).transpose(0, 2, 1, 3)
    nparts, eparts = _sc_segsum(x, x_batch.astype(jnp.int32), ea4,
                                edge_attr_batch.astype(jnp.int32))
    w0e = W0[:, :D_EDGE].T
    w0n = W0[:, D_EDGE:D_EDGE + D_FEAT].T
    w0u = W0[:, D_EDGE + D_FEAT:].T
    return _mlp(nparts, eparts, u, w0e, w0n, w0u, b0.reshape(1, -1),
                W1.T, b1.reshape(1, -1), W_out.T, b_out.reshape(1, -1))


# speedup vs baseline: 1.0341x; 1.0341x over previous
import jax
import jax.numpy as jnp
from jax import lax
from jax.experimental import pallas as pl
from jax.experimental.pallas import tpu as pltpu
from jax.experimental.pallas import tpu_sc as plsc

N_NODES = 100000
N_EDGES = 1600000
D_FEAT = 128
D_EDGE = 16
N_SEG = 256
N_CORES = 2
N_SUB = 16
N_WORKERS = N_CORES * N_SUB

NCH = 160
NCHUNKS = N_NODES // NCH
MAXN = 20
STR = 16
TC_TOT = N_EDGES // 128
TCC = 10
ECHUNKS = TC_TOT // TCC
MAXE = 40
EV = TCC * 128 // 16


def _hsum16(v):
    idx = lax.iota(jnp.int32, 16)
    for k in (8, 4, 2, 1):
        v = v + v.at[idx ^ k].get(mode="promise_in_bounds", unique_indices=True)
    return v[0]


def _sc_segsum(x, xb, ea4, eb):
    mesh = plsc.VectorSubcoreMesh(core_axis_name="c", subcore_axis_name="s")

    def body(x_hbm, xb_hbm, ea_hbm, eb_hbm, nout, eout,
             ebufA, ebufB, eidxA, eidxB, esemA, esemB,
             nbufA, nbufB, nidxA, nidxB, nsemA, nsemB,
             eacc, ssem, nsacc):
        c = lax.axis_index("c")
        s = lax.axis_index("s")
        w = c * N_SUB + s
        zero16 = jnp.zeros((16,), jnp.float32)

        @pl.when(s == 0)
        def _():
            def zrow(i, _):
                for j in range(D_FEAT // 16):
                    nbufA[i, pl.ds(j * 16, 16)] = zero16
                    nbufB[i, pl.ds(j * 16, 16)] = zero16
                return 0
            lax.fori_loop(0, 128, zrow, 0)
            pltpu.sync_copy(nbufA.at[pl.ds(0, 128)], nsacc.at[pl.ds(0, 128)])
            pltpu.sync_copy(nbufB.at[pl.ds(0, 128)], nsacc.at[pl.ds(128, 128)])

        def zerow(i, _):
            for j in range(N_SEG // 16):
                eacc[i, pl.ds(j * 16, 16)] = zero16
            return 0
        lax.fori_loop(0, D_EDGE, zerow, 0)

        plsc.subcore_barrier()

        egbase = jnp.where(w < 2, w * 40, 80 + (w - 2) * 39)
        ecount = jnp.where(w < 2, 40, 39)

        def e_start(ci, buf, idx, sem):
            @pl.when(ci < ecount)
            def _():
                tc0 = (egbase + ci) * TCC
                pltpu.async_copy(ea_hbm.at[0, pl.ds(tc0, TCC)], buf.at[0], sem)
                pltpu.async_copy(ea_hbm.at[1, pl.ds(tc0, TCC)], buf.at[1], sem)
                pltpu.async_copy(eb_hbm.at[pl.ds(tc0 * 128, TCC * 128)], idx, sem)

        def e_wait(buf, idx, sem):
            pltpu.make_async_copy(ea_hbm.at[0, pl.ds(0, TCC)], buf.at[0], sem).wait()
            pltpu.make_async_copy(ea_hbm.at[1, pl.ds(0, TCC)], buf.at[1], sem).wait()
            pltpu.make_async_copy(eb_hbm.at[pl.ds(0, TCC * 128)], idx, sem).wait()

        def e_compute(buf, idx):
            first = idx[pl.ds(0, 16)][0]
            last = idx[pl.ds(TCC * 128 - 16, 16)][15]

            def per_seg(sg, _):
                def acc_v(v, vaccs):
                    iv = idx[pl.ds(v * 16, 16)]
                    m = iv == sg
                    tc = v // 8
                    lane0 = (v % 8) * 16
                    out = []
                    for tr in range(2):
                        for fr in range(8):
                            d = buf[tr, tc, fr, pl.ds(lane0, 16)]
                            out.append(vaccs[tr * 8 + fr]
                                       + jnp.where(m, d, 0.0))
                    return tuple(out)
                vaccs = lax.fori_loop(0, EV, acc_v,
                                      tuple(zero16 for _ in range(D_EDGE)))
                sga = (sg // 16) * 16
                lane_m = lax.iota(jnp.int32, 16) == (sg - sga)
                for f in range(D_EDGE):
                    tot = _hsum16(vaccs[f])
                    row = eacc[f, pl.ds(sga, 16)]
                    eacc[f, pl.ds(sga, 16)] = row + jnp.where(lane_m, tot, 0.0)
                return 0

            lax.fori_loop(first, last + 1, per_seg, 0)

        e_start(0, ebufA, eidxA, esemA)

        def e_body(ci2, _):
            ci0 = ci2 * 2
            e_start(ci0 + 1, ebufB, eidxB, esemB)
            @pl.when(ci0 < ecount)
            def _():
                e_wait(ebufA, eidxA, esemA)
                e_compute(ebufA, eidxA)
            e_start(ci0 + 2, ebufA, eidxA, esemA)
            @pl.when(ci0 + 1 < ecount)
            def _():
                e_wait(ebufB, eidxB, esemB)
                e_compute(ebufB, eidxB)
            return 0

        lax.fori_loop(0, MAXE // 2, e_body, 0)

        ngbase = jnp.where(w < 17, w * 20, 340 + (w - 17) * 19)
        ncount = jnp.where(w < 17, 20, 19)

        def n_start(ci, buf, idx, sem):
            @pl.when(ci < ncount)
            def _():
                rb = (ngbase + ci) * NCH
                pltpu.async_copy(x_hbm.at[pl.ds(rb, NCH)], buf, sem)
                pltpu.async_copy(xb_hbm.at[pl.ds(rb, NCH)], idx, sem)

        def n_wait(buf, idx, sem):
            pltpu.make_async_copy(x_hbm.at[pl.ds(0, NCH)], buf, sem).wait()
            pltpu.make_async_copy(xb_hbm.at[pl.ds(0, NCH)], idx, sem).wait()

        def n_compute(buf, idx):
            descs = []
            for j in range(NCH // STR):
                iv = idx[pl.ds(j * STR, STR)]
                descs.append(pltpu.async_copy(
                    buf.at[pl.ds(j * STR, STR)], nsacc.at[iv], ssem, add=True))
            for d in descs:
                d.wait()

        n_start(0, nbufA, nidxA, nsemA)

        def n_body(ci2, _):
            ci0 = ci2 * 2
            n_start(ci0 + 1, nbufB, nidxB, nsemB)
            @pl.when(ci0 < ncount)
            def _():
                n_wait(nbufA, nidxA, nsemA)
                n_compute(nbufA, nidxA)
            n_start(ci0 + 2, nbufA, nidxA, nsemA)
            @pl.when(ci0 + 1 < ncount)
            def _():
                n_wait(nbufB, nidxB, nsemB)
                n_compute(nbufB, nidxB)
            return 0

        lax.fori_loop(0, MAXN // 2, n_body, 0)

        plsc.subcore_barrier()

        @pl.when(s == 0)
        def _():
            pltpu.sync_copy(nsacc, nout.at[c])
        pltpu.sync_copy(eacc, eout.at[w])

    f = pl.kernel(
        body,
        out_type=(jax.ShapeDtypeStruct((N_CORES, N_SEG, D_FEAT), jnp.float32),
                  jax.ShapeDtypeStruct((N_WORKERS, D_EDGE, N_SEG), jnp.float32)),
        mesh=mesh,
        compiler_params=pltpu.CompilerParams(use_tc_tiling_on_sc=False),
        scratch_types=[
            pltpu.VMEM((2, TCC, 8, 128), jnp.float32),
            pltpu.VMEM((2, TCC, 8, 128), jnp.float32),
            pltpu.VMEM((TCC * 128,), jnp.int32),
            pltpu.VMEM((TCC * 128,), jnp.int32),
            pltpu.SemaphoreType.DMA,
            pltpu.SemaphoreType.DMA,
            pltpu.VMEM((NCH, D_FEAT), jnp.float32),
            pltpu.VMEM((NCH, D_FEAT), jnp.float32),
            pltpu.VMEM((NCH,), jnp.int32),
            pltpu.VMEM((NCH,), jnp.int32),
            pltpu.SemaphoreType.DMA,
            pltpu.SemaphoreType.DMA,
            pltpu.VMEM((D_EDGE, N_SEG), jnp.float32),
            pltpu.SemaphoreType.DMA,
            pltpu.VMEM_SHARED((N_SEG, D_FEAT), jnp.float32),
        ],
    )
    return f(x, xb, ea4, eb)


def _mlp(nparts, eparts, u, w0e, w0n, w0u, b0, w1, b1, wo, bo):
    def body(np_ref, ep_ref, u_ref, w0e_ref, w0n_ref, w0u_ref, b0_ref,
             w1_ref, b1_ref, wo_ref, bo_ref, o_ref):
        n = np_ref[0] + np_ref[1]
        e = jnp.sum(ep_ref[...], axis=0)
        he = lax.dot_general(e, w0e_ref[...], (((0,), (0,)), ((), ())),
                             preferred_element_type=jnp.float32)
        h = (he
             + jnp.dot(n, w0n_ref[...], preferred_element_type=jnp.float32)
             + jnp.dot(u_ref[...], w0u_ref[...], preferred_element_type=jnp.float32)
             + b0_ref[...])
        h = jnp.where(h >= 0, h, 0.2 * h)
        h = jnp.dot(h, w1_ref[...], preferred_element_type=jnp.float32) + b1_ref[...]
        h = jnp.where(h >= 0, h, 0.2 * h)
        o_ref[...] = (jnp.dot(h, wo_ref[...], preferred_element_type=jnp.float32)
                      + bo_ref[...])

    return pl.pallas_call(
        body,
        out_shape=jax.ShapeDtypeStruct((N_SEG, 128), jnp.float32),
    )(nparts, eparts, u, w0e, w0n, w0u, b0, w1, b1, wo, bo)


def kernel(x, edge_index, edge_attr, u, x_batch, edge_attr_batch,
           W0, b0, W1, b1, W_out, b_out):
    ea4 = edge_attr.T.reshape(2, 8, TC_TOT, 128).transpose(0, 2, 1, 3)
    nparts, eparts = _sc_segsum(x, x_batch.astype(jnp.int32), ea4,
                                edge_attr_batch.astype(jnp.int32))
    w0e = W0[:, :D_EDGE].T
    w0n = W0[:, D_EDGE:D_EDGE + D_FEAT].T
    w0u = W0[:, D_EDGE + D_FEAT:].T
    return _mlp(nparts, eparts, u, w0e, w0n, w0u, b0.reshape(1, -1),
                W1.T, b1.reshape(1, -1), W_out.T, b_out.reshape(1, -1))
